# baseline (speedup 1.0000x reference)
import jax
import jax.numpy as jnp
from jax.experimental import pallas as pl


def _body(x_ref, w_ref, b_ref, o_ref):
    l = jnp.dot(x_ref[...], w_ref[...], preferred_element_type=jnp.float32)
    l = l + b_ref[...]

    def col(k):
        return l[:, k:k + 1]

    part = col(1) > col(0)
    bv = col(2)
    bi = jnp.zeros_like(bv, dtype=jnp.int32)
    t = col(3) > bv
    bi = jnp.where(t, 1, bi)
    bv = jnp.where(t, col(3), bv)
    t = col(4) > bv
    bi = jnp.where(t, 2, bi)
    rv = col(5)
    ri = jnp.zeros_like(bv, dtype=jnp.int32)
    for k in range(1, 8):
        t = col(5 + k) > rv
        ri = jnp.where(t, k, ri)
        rv = jnp.where(t, col(5 + k), rv)
    a0 = col(13) >= col(14)

    branch = jnp.where(bi == 0, 3, jnp.where(bi == 1, ri + 1, jnp.where(a0, 4, 6)))
    final = jnp.where(part, branch, 0).astype(jnp.int32)
    o_ref[...] = final[:, 0]


def kernel(x, W1, b1, W2, b2, W3r, b3r, W3a, b3a):
    batch = x.shape[0]
    xf = x.reshape(batch, -1)
    d = xf.shape[1]
    W = jnp.concatenate([W1, W2, W3r, W3a], axis=1)
    b = jnp.concatenate([b1, b2, b3r, b3a], axis=0)
    W = jnp.pad(W, ((0, 0), (0, 128 - W.shape[1])))
    b = jnp.pad(b, ((0, 128 - b.shape[0]),)).reshape(1, 128)

    grid = 8
    bs = batch // grid
    out = pl.pallas_call(
        _body,
        grid=(grid,),
        in_specs=[
            pl.BlockSpec((bs, d), lambda i: (i, 0)),
            pl.BlockSpec((d, 128), lambda i: (0, 0)),
            pl.BlockSpec((1, 128), lambda i: (0, 0)),
        ],
        out_specs=pl.BlockSpec((bs,), lambda i: (i,)),
        out_shape=jax.ShapeDtypeStruct((batch,), jnp.int32),
    )(xf, W, b)
    return out


# speedup vs baseline: 2.2517x; 2.2517x over previous
import jax
import jax.numpy as jnp
from jax.experimental import pallas as pl


def _body(x_ref, w_ref, b_ref, o_ref):
    l = jnp.dot(x_ref[...], w_ref[...], preferred_element_type=jnp.float32)
    l = l + b_ref[...]
    lt = l.T

    def row(k):
        return lt[k:k + 1, :]

    part = row(1) > row(0)
    bv = row(2)
    bi = jnp.zeros_like(bv, dtype=jnp.int32)
    t = row(3) > bv
    bi = jnp.where(t, 1, bi)
    bv = jnp.where(t, row(3), bv)
    t = row(4) > bv
    bi = jnp.where(t, 2, bi)
    rv = row(5)
    ri = jnp.zeros_like(bv, dtype=jnp.int32)
    for k in range(1, 8):
        t = row(5 + k) > rv
        ri = jnp.where(t, k, ri)
        rv = jnp.where(t, row(5 + k), rv)
    a0 = row(13) >= row(14)

    branch = jnp.where(bi == 0, 3, jnp.where(bi == 1, ri + 1, jnp.where(a0, 4, 6)))
    final = jnp.where(part, branch, 0).astype(jnp.int32)
    o_ref[...] = final.reshape(o_ref.shape)


def kernel(x, W1, b1, W2, b2, W3r, b3r, W3a, b3a):
    batch = x.shape[0]
    xf = x.reshape(batch, -1)
    d = xf.shape[1]
    W = jnp.concatenate([W1, W2, W3r, W3a], axis=1)
    b = jnp.concatenate([b1, b2, b3r, b3a], axis=0)
    W = jnp.pad(W, ((0, 0), (0, 128 - W.shape[1])))
    b = jnp.pad(b, ((0, 128 - b.shape[0]),)).reshape(1, 128)

    grid = 8
    bs = batch // grid
    out = pl.pallas_call(
        _body,
        grid=(grid,),
        in_specs=[
            pl.BlockSpec((bs, d), lambda i: (i, 0)),
            pl.BlockSpec((d, 128), lambda i: (0, 0)),
            pl.BlockSpec((1, 128), lambda i: (0, 0)),
        ],
        out_specs=pl.BlockSpec((1, 1, bs), lambda i: (i, 0, 0)),
        out_shape=jax.ShapeDtypeStruct((grid, 1, bs), jnp.int32),
    )(xf, W, b)
    return out.reshape(batch)
